# baseline (speedup 1.0000x reference)
import functools
import math

import jax
import jax.numpy as jnp
from jax.experimental import pallas as pl
from jax.experimental.pallas import tpu as pltpu

_NUM_CLASSES = 81
_ALPHA = 1.0
_NEG_POS_RATIO = 3.0
_NEGATIVES_FOR_HARD = 100.0
_LOG_EPS = math.log(1e-7)


def _pass1_body(n_total, block_n, yt_ref, xl_ref, xc_ref, conf_ref, key_ref,
                stats_ref):
    c = pl.program_id(1)
    yt = yt_ref[0]
    xl = xl_ref[0]
    xc = xc_ref[0]

    rows = c * block_n + jax.lax.broadcasted_iota(jnp.int32, (block_n, 1), 0)
    valid = rows < n_total

    yt = jnp.where(valid, yt, 0.0)
    xl = jnp.where(valid, xl, 0.0)
    xc = jnp.where(valid, xc, 0.0)

    m = jnp.max(xc, axis=1, keepdims=True)
    e = jnp.exp(xc - m)
    s = jnp.sum(e, axis=1, keepdims=True)
    logz = m + jnp.log(s)
    lp = jnp.maximum(xc - logz, _LOG_EPS)

    ycls = yt[:, 4:4 + _NUM_CLASSES]
    conf_loss = -jnp.sum(ycls * lp, axis=1, keepdims=True)

    d = yt[:, 0:4] - xl
    ad = jnp.abs(d)
    l1 = jnp.where(ad < 1.0, 0.5 * d * d, ad - 0.5)
    loc_loss = jnp.sum(l1, axis=1, keepdims=True)

    pos = yt[:, 4 + _NUM_CLASSES:5 + _NUM_CLASSES]

    key = (1.0 - e[:, 0:1] / s) * (1.0 - pos)
    key = jnp.where(valid, jnp.maximum(key, 0.0), -1.0)

    conf_ref[0] = jnp.where(valid, conf_loss, 0.0)
    key_ref[0] = key

    pc = jnp.sum(conf_loss * pos)
    plc = jnp.sum(loc_loss * pos)
    npos = jnp.sum(pos)
    lane = jax.lax.broadcasted_iota(jnp.int32, (1, 1, 8), 2)
    vec = jnp.where(lane == 0, pc, jnp.where(lane == 1, plc,
                                             jnp.where(lane == 2, npos, 0.0)))
    prev = jnp.where(c == 0, jnp.zeros_like(vec), stats_ref[...])
    stats_ref[...] = prev + vec


def _select_body(n_rows, k_ref, key_ref, conf_ref, out_ref):
    keys = key_ref[...]
    confs = conf_ref[...]
    bits = jax.lax.bitcast_convert_type(keys, jnp.int32)
    k = k_ref[0]

    def step(_, carry):
        lo, hi = carry
        mid = jax.lax.div(lo + hi, 2)
        cnt = jnp.sum((bits >= mid).astype(jnp.int32))
        big = cnt >= k
        return (jnp.where(big, mid, lo), jnp.where(big, hi, mid))

    lo, _ = jax.lax.fori_loop(0, 31, step, (jnp.int32(0), jnp.int32(0x7F800000)))
    gt = bits > lo
    eq = bits == lo
    sum_gt = jnp.sum(jnp.where(gt, confs, 0.0))
    cnt_gt = jnp.sum(gt.astype(jnp.float32))
    sum_eq = jnp.sum(jnp.where(eq, confs, 0.0))
    cnt_eq = jnp.sum(eq.astype(jnp.float32))
    rem = jnp.maximum(k.astype(jnp.float32) - cnt_gt, 0.0)
    rem = jnp.minimum(rem, cnt_eq)
    out_ref[0] = sum_gt + rem * sum_eq / jnp.maximum(cnt_eq, 1.0)


def kernel(y_true, y_pred_loc, y_pred_conf):
    b, n, c = y_pred_conf.shape
    block_n = 4096
    nblocks = (n + block_n - 1) // block_n

    conf_map, key_map, stats = pl.pallas_call(
        functools.partial(_pass1_body, n, block_n),
        grid=(b, nblocks),
        in_specs=[
            pl.BlockSpec((1, block_n, y_true.shape[2]), lambda i, j: (i, j, 0)),
            pl.BlockSpec((1, block_n, 4), lambda i, j: (i, j, 0)),
            pl.BlockSpec((1, block_n, c), lambda i, j: (i, j, 0)),
        ],
        out_specs=[
            pl.BlockSpec((1, block_n, 1), lambda i, j: (i, j, 0)),
            pl.BlockSpec((1, block_n, 1), lambda i, j: (i, j, 0)),
            pl.BlockSpec((1, 1, 8), lambda i, j: (i, 0, 0)),
        ],
        out_shape=[
            jax.ShapeDtypeStruct((b, n, 1), jnp.float32),
            jax.ShapeDtypeStruct((b, n, 1), jnp.float32),
            jax.ShapeDtypeStruct((b, 1, 8), jnp.float32),
        ],
    )(y_true, y_pred_loc, y_pred_conf)

    pos_conf = jnp.sum(stats[:, 0, 0])
    pos_loc = jnp.sum(stats[:, 0, 1])
    num_pos = stats[:, 0, 2]

    num_neg = jnp.minimum(_NEG_POS_RATIO * num_pos, n - num_pos)
    has_min = jnp.sum((num_neg > 0).astype(jnp.float32))
    nnb = jnp.where(has_min > 0, jnp.sum(num_neg),
                    jnp.asarray(_NEGATIVES_FOR_HARD, jnp.float32))
    kk = jnp.floor(nnb).astype(jnp.int32).reshape(1)

    total = b * n
    n_rows = (total + 127) // 128
    pad = n_rows * 128 - total
    keys_flat = key_map.reshape(-1)
    confs_flat = conf_map.reshape(-1)
    if pad:
        keys_flat = jnp.concatenate(
            [keys_flat, jnp.full((pad,), -1.0, jnp.float32)])
        confs_flat = jnp.concatenate([confs_flat, jnp.zeros((pad,), jnp.float32)])
    keys2 = keys_flat.reshape(n_rows, 128)
    confs2 = confs_flat.reshape(n_rows, 128)

    neg = pl.pallas_call(
        functools.partial(_select_body, n_rows),
        in_specs=[
            pl.BlockSpec(memory_space=pltpu.SMEM),
            pl.BlockSpec((n_rows, 128), lambda: (0, 0)),
            pl.BlockSpec((n_rows, 128), lambda: (0, 0)),
        ],
        out_specs=pl.BlockSpec(memory_space=pltpu.SMEM),
        out_shape=jax.ShapeDtypeStruct((1,), jnp.float32),
    )(kk, keys2, confs2)[0]

    denom = jnp.sum(jnp.where(num_pos != 0, num_pos, 1.0))
    return (pos_conf + neg + _ALPHA * pos_loc) / denom


# speedup vs baseline: 1.0584x; 1.0584x over previous
import functools
import math

import jax
import jax.numpy as jnp
from jax.experimental import pallas as pl
from jax.experimental.pallas import tpu as pltpu

_NUM_CLASSES = 81
_ALPHA = 1.0
_NEG_POS_RATIO = 3.0
_NEGATIVES_FOR_HARD = 100.0
_LOG_EPS = math.log(1e-7)


def _pass1_body(n_total, block_n, yt_ref, xl_ref, xc_ref, conf_ref, key_ref,
                stats_ref):
    c = pl.program_id(1)
    yt = yt_ref[0]
    xl = xl_ref[0]
    xc = xc_ref[0]

    rows = c * block_n + jax.lax.broadcasted_iota(jnp.int32, (block_n, 1), 0)
    valid = rows < n_total

    yt = jnp.where(valid, yt, 0.0)
    xl = jnp.where(valid, xl, 0.0)
    xc = jnp.where(valid, xc, 0.0)

    m = jnp.max(xc, axis=1, keepdims=True)
    e = jnp.exp(xc - m)
    s = jnp.sum(e, axis=1, keepdims=True)
    logz = m + jnp.log(s)
    lp = jnp.maximum(xc - logz, _LOG_EPS)

    ycls = yt[:, 4:4 + _NUM_CLASSES]
    conf_loss = -jnp.sum(ycls * lp, axis=1, keepdims=True)

    d = yt[:, 0:4] - xl
    ad = jnp.abs(d)
    l1 = jnp.where(ad < 1.0, 0.5 * d * d, ad - 0.5)
    loc_loss = jnp.sum(l1, axis=1, keepdims=True)

    pos = yt[:, 4 + _NUM_CLASSES:5 + _NUM_CLASSES]

    key = (1.0 - e[:, 0:1] / s) * (1.0 - pos)
    key = jnp.where(valid, jnp.maximum(key, 0.0), -1.0)

    rows128 = block_n // 128
    conf_ref[...] = jnp.reshape(jnp.where(valid, conf_loss, 0.0),
                                (rows128, 128))
    key_ref[...] = jnp.reshape(key, (rows128, 128))

    pc = jnp.sum(conf_loss * pos)
    plc = jnp.sum(loc_loss * pos)
    npos = jnp.sum(pos)
    lane = jax.lax.broadcasted_iota(jnp.int32, (1, 1, 8), 2)
    vec = jnp.where(lane == 0, pc, jnp.where(lane == 1, plc,
                                             jnp.where(lane == 2, npos, 0.0)))
    prev = jnp.where(c == 0, jnp.zeros_like(vec), stats_ref[...])
    stats_ref[...] = prev + vec


def _select_body(n_rows, k_ref, key_ref, conf_ref, out_ref):
    keys = key_ref[...]
    confs = conf_ref[...]
    bits = jax.lax.bitcast_convert_type(keys, jnp.int32)
    k = k_ref[0]

    def step(_, carry):
        lo, hi = carry
        mid = jax.lax.div(lo + hi, 2)
        cnt = jnp.sum((bits >= mid).astype(jnp.int32))
        big = cnt >= k
        return (jnp.where(big, mid, lo), jnp.where(big, hi, mid))

    lo, _ = jax.lax.fori_loop(0, 31, step, (jnp.int32(0), jnp.int32(0x7F800000)))
    gt = bits > lo
    eq = bits == lo
    sum_gt = jnp.sum(jnp.where(gt, confs, 0.0))
    cnt_gt = jnp.sum(gt.astype(jnp.float32))
    sum_eq = jnp.sum(jnp.where(eq, confs, 0.0))
    cnt_eq = jnp.sum(eq.astype(jnp.float32))
    rem = jnp.maximum(k.astype(jnp.float32) - cnt_gt, 0.0)
    rem = jnp.minimum(rem, cnt_eq)
    out_ref[0] = sum_gt + rem * sum_eq / jnp.maximum(cnt_eq, 1.0)


def kernel(y_true, y_pred_loc, y_pred_conf):
    b, n, c = y_pred_conf.shape
    block_n = 4096
    nblocks = (n + block_n - 1) // block_n

    rows128 = block_n // 128
    n_rows = b * nblocks * rows128
    conf2, keys2, stats = pl.pallas_call(
        functools.partial(_pass1_body, n, block_n),
        grid=(b, nblocks),
        in_specs=[
            pl.BlockSpec((1, block_n, y_true.shape[2]), lambda i, j: (i, j, 0)),
            pl.BlockSpec((1, block_n, 4), lambda i, j: (i, j, 0)),
            pl.BlockSpec((1, block_n, c), lambda i, j: (i, j, 0)),
        ],
        out_specs=[
            pl.BlockSpec((rows128, 128),
                         lambda i, j, nb=nblocks: (i * nb + j, 0)),
            pl.BlockSpec((rows128, 128),
                         lambda i, j, nb=nblocks: (i * nb + j, 0)),
            pl.BlockSpec((1, 1, 8), lambda i, j: (i, 0, 0)),
        ],
        out_shape=[
            jax.ShapeDtypeStruct((n_rows, 128), jnp.float32),
            jax.ShapeDtypeStruct((n_rows, 128), jnp.float32),
            jax.ShapeDtypeStruct((b, 1, 8), jnp.float32),
        ],
    )(y_true, y_pred_loc, y_pred_conf)

    pos_conf = jnp.sum(stats[:, 0, 0])
    pos_loc = jnp.sum(stats[:, 0, 1])
    num_pos = stats[:, 0, 2]

    num_neg = jnp.minimum(_NEG_POS_RATIO * num_pos, n - num_pos)
    has_min = jnp.sum((num_neg > 0).astype(jnp.float32))
    nnb = jnp.where(has_min > 0, jnp.sum(num_neg),
                    jnp.asarray(_NEGATIVES_FOR_HARD, jnp.float32))
    kk = jnp.floor(nnb).astype(jnp.int32).reshape(1)

    neg = pl.pallas_call(
        functools.partial(_select_body, n_rows),
        in_specs=[
            pl.BlockSpec(memory_space=pltpu.SMEM),
            pl.BlockSpec((n_rows, 128), lambda: (0, 0)),
            pl.BlockSpec((n_rows, 128), lambda: (0, 0)),
        ],
        out_specs=pl.BlockSpec(memory_space=pltpu.SMEM),
        out_shape=jax.ShapeDtypeStruct((1,), jnp.float32),
    )(kk, keys2, conf2)[0]

    denom = jnp.sum(jnp.where(num_pos != 0, num_pos, 1.0))
    return (pos_conf + neg + _ALPHA * pos_loc) / denom


# speedup vs baseline: 9.1975x; 8.6898x over previous
import functools
import math

import jax
import jax.numpy as jnp
from jax.experimental import pallas as pl
from jax.experimental.pallas import tpu as pltpu

_NUM_CLASSES = 81
_ALPHA = 1.0
_NEG_POS_RATIO = 3.0
_NEGATIVES_FOR_HARD = 100.0
_LOG_EPS = math.log(1e-7)


def _pass1_body(n_total, block_n, yt_ref, xl_ref, xc_ref, conf_ref, key_ref,
                stats_ref):
    j = pl.program_id(0)
    yt = yt_ref[...]
    xl = xl_ref[...]
    xc = xc_ref[...]

    lanes = j * block_n + jax.lax.broadcasted_iota(
        jnp.int32, (1, block_n), 1)
    valid = lanes < n_total

    m = jnp.max(xc, axis=0)
    e = jnp.exp(xc - m)
    s = jnp.sum(e, axis=0)
    logz = m + jnp.log(s)
    lp = jnp.maximum(xc - logz, _LOG_EPS)

    ycls = yt[4:4 + _NUM_CLASSES]
    conf_loss = -jnp.sum(ycls * lp, axis=0)

    d = yt[0:4] - xl
    ad = jnp.abs(d)
    l1 = jnp.where(ad < 1.0, 0.5 * d * d, ad - 0.5)
    loc_loss = jnp.sum(l1, axis=0)

    pos = yt[4 + _NUM_CLASSES]

    key = jnp.maximum((1.0 - e[0] / s) * (1.0 - pos), 0.0)

    conf_ref[...] = conf_loss
    key_ref[...] = key

    posv = jnp.where(valid, pos, 0.0)
    confv = jnp.where(valid, conf_loss, 0.0)
    locv = jnp.where(valid, loc_loss, 0.0)
    pc = jnp.sum(confv * posv, axis=1, keepdims=True)
    plc = jnp.sum(locv * posv, axis=1, keepdims=True)
    npos = jnp.sum(posv, axis=1, keepdims=True)
    lane = jax.lax.broadcasted_iota(jnp.int32, (yt.shape[1], 8), 1)
    vec = jnp.where(lane == 0, pc, jnp.where(lane == 1, plc,
                                             jnp.where(lane == 2, npos, 0.0)))
    prev = jnp.where(j == 0, jnp.zeros_like(vec), stats_ref[...])
    stats_ref[...] = prev + vec


def _select_body(k_ref, key_ref, conf_ref, out_ref):
    keys = key_ref[...]
    confs = conf_ref[...]
    bits = jax.lax.bitcast_convert_type(keys, jnp.int32)
    k = k_ref[0]

    def step(_, carry):
        lo, hi = carry
        mid = jax.lax.div(lo + hi, 2)
        cnt = jnp.sum((bits >= mid).astype(jnp.int32))
        big = cnt >= k
        return (jnp.where(big, mid, lo), jnp.where(big, hi, mid))

    lo, _ = jax.lax.fori_loop(0, 31, step, (jnp.int32(0), jnp.int32(0x7F800000)))
    gt = bits > lo
    eq = bits == lo
    sum_gt = jnp.sum(jnp.where(gt, confs, 0.0))
    cnt_gt = jnp.sum(gt.astype(jnp.float32))
    sum_eq = jnp.sum(jnp.where(eq, confs, 0.0))
    cnt_eq = jnp.sum(eq.astype(jnp.float32))
    rem = jnp.maximum(k.astype(jnp.float32) - cnt_gt, 0.0)
    rem = jnp.minimum(rem, cnt_eq)
    out_ref[0] = sum_gt + rem * sum_eq / jnp.maximum(cnt_eq, 1.0)


def kernel(y_true, y_pred_loc, y_pred_conf):
    b, n, c = y_pred_conf.shape
    nf = y_true.shape[2]
    block_n = 512
    nblocks = (n + block_n - 1) // block_n

    yt_t = jnp.transpose(y_true, (2, 0, 1))
    xl_t = jnp.transpose(y_pred_loc, (2, 0, 1))
    xc_t = jnp.transpose(y_pred_conf, (2, 0, 1))

    conf2, keys2, stats = pl.pallas_call(
        functools.partial(_pass1_body, n, block_n),
        grid=(nblocks,),
        in_specs=[
            pl.BlockSpec((nf, b, block_n), lambda j: (0, 0, j)),
            pl.BlockSpec((4, b, block_n), lambda j: (0, 0, j)),
            pl.BlockSpec((c, b, block_n), lambda j: (0, 0, j)),
        ],
        out_specs=[
            pl.BlockSpec((b, block_n), lambda j: (0, j)),
            pl.BlockSpec((b, block_n), lambda j: (0, j)),
            pl.BlockSpec((b, 8), lambda j: (0, 0)),
        ],
        out_shape=[
            jax.ShapeDtypeStruct((b, n), jnp.float32),
            jax.ShapeDtypeStruct((b, n), jnp.float32),
            jax.ShapeDtypeStruct((b, 8), jnp.float32),
        ],
    )(yt_t, xl_t, xc_t)

    pos_conf = jnp.sum(stats[:, 0])
    pos_loc = jnp.sum(stats[:, 1])
    num_pos = stats[:, 2]

    num_neg = jnp.minimum(_NEG_POS_RATIO * num_pos, n - num_pos)
    has_min = jnp.sum((num_neg > 0).astype(jnp.float32))
    nnb = jnp.where(has_min > 0, jnp.sum(num_neg),
                    jnp.asarray(_NEGATIVES_FOR_HARD, jnp.float32))
    kk = jnp.floor(nnb).astype(jnp.int32).reshape(1)

    neg = pl.pallas_call(
        _select_body,
        in_specs=[
            pl.BlockSpec(memory_space=pltpu.SMEM),
            pl.BlockSpec((b, n), lambda: (0, 0)),
            pl.BlockSpec((b, n), lambda: (0, 0)),
        ],
        out_specs=pl.BlockSpec(memory_space=pltpu.SMEM),
        out_shape=jax.ShapeDtypeStruct((1,), jnp.float32),
    )(kk, keys2, conf2)[0]

    denom = jnp.sum(jnp.where(num_pos != 0, num_pos, 1.0))
    return (pos_conf + neg + _ALPHA * pos_loc) / denom


# speedup vs baseline: 9.6386x; 1.0480x over previous
import functools
import math

import jax
import jax.numpy as jnp
from jax.experimental import pallas as pl
from jax.experimental.pallas import tpu as pltpu

_NUM_CLASSES = 81
_ALPHA = 1.0
_NEG_POS_RATIO = 3.0
_NEGATIVES_FOR_HARD = 100.0
_LOG_EPS = math.log(1e-7)


def _body(n_total, block_n, nblocks, yt_ref, xl_ref, xc_ref, out_ref,
          conf_scr, key_scr, stats_ref):
    j = pl.program_id(0)
    yt = yt_ref[...]
    xl = xl_ref[...]
    xc = xc_ref[...]

    lanes = j * block_n + jax.lax.broadcasted_iota(jnp.int32, (1, block_n), 1)
    valid = lanes < n_total

    m = jnp.max(xc, axis=0)
    e = jnp.exp(xc - m)
    s = jnp.sum(e, axis=0)
    logz = m + jnp.log(s)
    lp = jnp.maximum(xc - logz, _LOG_EPS)

    ycls = yt[4:4 + _NUM_CLASSES]
    conf_loss = -jnp.sum(ycls * lp, axis=0)

    d = yt[0:4] - xl
    ad = jnp.abs(d)
    l1 = jnp.where(ad < 1.0, 0.5 * d * d, ad - 0.5)
    loc_loss = jnp.sum(l1, axis=0)

    pos = yt[4 + _NUM_CLASSES]

    key = jnp.maximum((1.0 - e[0] / s) * (1.0 - pos), 0.0)
    key_scr[:, pl.ds(j * block_n, block_n)] = jnp.where(valid, key, -1.0)
    conf_scr[:, pl.ds(j * block_n, block_n)] = conf_loss

    posv = jnp.where(valid, pos, 0.0)
    confv = jnp.where(valid, conf_loss, 0.0)
    locv = jnp.where(valid, loc_loss, 0.0)
    pc = jnp.sum(confv * posv, axis=1, keepdims=True)
    plc = jnp.sum(locv * posv, axis=1, keepdims=True)
    npos = jnp.sum(posv, axis=1, keepdims=True)
    lane = jax.lax.broadcasted_iota(jnp.int32, (yt.shape[1], 8), 1)
    vec = jnp.where(lane == 0, pc, jnp.where(lane == 1, plc,
                                             jnp.where(lane == 2, npos, 0.0)))
    prev = jnp.where(j == 0, jnp.zeros_like(vec), stats_ref[...])
    stats_ref[...] = prev + vec

    @pl.when(j == nblocks - 1)
    def _selection():
        st = stats_ref[...]
        npos_c = st[:, 2:3]
        num_neg = jnp.minimum(_NEG_POS_RATIO * npos_c, n_total - npos_c)
        has_min = jnp.sum((num_neg > 0).astype(jnp.float32))
        nnb = jnp.where(has_min > 0, jnp.sum(num_neg),
                        jnp.float32(_NEGATIVES_FOR_HARD))
        k = jnp.floor(nnb).astype(jnp.int32)

        bits = jax.lax.bitcast_convert_type(key_scr[...], jnp.int32)
        confs = conf_scr[...]

        def step(_, carry):
            lo, hi = carry
            mid = jax.lax.div(lo + hi, 2)
            cnt = jnp.sum((bits >= mid).astype(jnp.int32))
            big = cnt >= k
            return (jnp.where(big, mid, lo), jnp.where(big, hi, mid))

        lo, _ = jax.lax.fori_loop(0, 31, step,
                                  (jnp.int32(0), jnp.int32(0x7F800000)))
        gt = bits > lo
        eq = bits == lo
        sum_gt = jnp.sum(jnp.where(gt, confs, 0.0))
        cnt_gt = jnp.sum(gt.astype(jnp.float32))
        sum_eq = jnp.sum(jnp.where(eq, confs, 0.0))
        cnt_eq = jnp.sum(eq.astype(jnp.float32))
        rem = jnp.maximum(k.astype(jnp.float32) - cnt_gt, 0.0)
        rem = jnp.minimum(rem, cnt_eq)
        neg = sum_gt + rem * sum_eq / jnp.maximum(cnt_eq, 1.0)

        pos_conf = jnp.sum(st[:, 0:1])
        pos_loc = jnp.sum(st[:, 1:2])
        denom = jnp.sum(jnp.where(npos_c != 0.0, npos_c, 1.0))
        out_ref[0] = (pos_conf + neg + _ALPHA * pos_loc) / denom


def kernel(y_true, y_pred_loc, y_pred_conf):
    b, n, c = y_pred_conf.shape
    nf = y_true.shape[2]
    block_n = 512
    nblocks = (n + block_n - 1) // block_n

    yt_t = jnp.transpose(y_true, (2, 0, 1))
    xl_t = jnp.transpose(y_pred_loc, (2, 0, 1))
    xc_t = jnp.transpose(y_pred_conf, (2, 0, 1))

    out = pl.pallas_call(
        functools.partial(_body, n, block_n, nblocks),
        grid=(nblocks,),
        in_specs=[
            pl.BlockSpec((nf, b, block_n), lambda j: (0, 0, j)),
            pl.BlockSpec((4, b, block_n), lambda j: (0, 0, j)),
            pl.BlockSpec((c, b, block_n), lambda j: (0, 0, j)),
        ],
        out_specs=pl.BlockSpec(memory_space=pltpu.SMEM),
        out_shape=jax.ShapeDtypeStruct((1,), jnp.float32),
        scratch_shapes=[
            pltpu.VMEM((b, nblocks * block_n), jnp.float32),
            pltpu.VMEM((b, nblocks * block_n), jnp.float32),
            pltpu.VMEM((b, 8), jnp.float32),
        ],
    )(yt_t, xl_t, xc_t)
    return out[0]


# speedup vs baseline: 9.9950x; 1.0370x over previous
import functools
import math

import jax
import jax.numpy as jnp
from jax.experimental import pallas as pl
from jax.experimental.pallas import tpu as pltpu

_NUM_CLASSES = 81
_ALPHA = 1.0
_NEG_POS_RATIO = 3.0
_NEGATIVES_FOR_HARD = 100.0
_LOG_EPS = math.log(1e-7)


def _body(n_total, block_n, nblocks, yt_ref, xl_ref, xc_ref, out_ref,
          conf_scr, key_scr, stats_ref):
    j = pl.program_id(0)
    yt = yt_ref[...]
    xl = xl_ref[...]
    xc = xc_ref[...]

    lanes = j * block_n + jax.lax.broadcasted_iota(jnp.int32, (1, block_n), 1)
    valid = lanes < n_total

    m = jnp.max(xc, axis=0)
    e = jnp.exp(xc - m)
    s = jnp.sum(e, axis=0)
    logz = m + jnp.log(s)
    lp = jnp.maximum(xc - logz, _LOG_EPS)

    ycls = yt[4:4 + _NUM_CLASSES]
    conf_loss = -jnp.sum(ycls * lp, axis=0)

    d = yt[0:4] - xl
    ad = jnp.abs(d)
    l1 = jnp.where(ad < 1.0, 0.5 * d * d, ad - 0.5)
    loc_loss = jnp.sum(l1, axis=0)

    pos = yt[4 + _NUM_CLASSES]

    key = jnp.maximum((1.0 - e[0] / s) * (1.0 - pos), 0.0)
    key_scr[:, pl.ds(j * block_n, block_n)] = jnp.where(valid, key, -1.0)
    conf_scr[:, pl.ds(j * block_n, block_n)] = conf_loss

    posv = jnp.where(valid, pos, 0.0)
    confv = jnp.where(valid, conf_loss, 0.0)
    locv = jnp.where(valid, loc_loss, 0.0)
    pc = jnp.sum(confv * posv, axis=1, keepdims=True)
    plc = jnp.sum(locv * posv, axis=1, keepdims=True)
    npos = jnp.sum(posv, axis=1, keepdims=True)
    lane = jax.lax.broadcasted_iota(jnp.int32, (yt.shape[1], 8), 1)
    vec = jnp.where(lane == 0, pc, jnp.where(lane == 1, plc,
                                             jnp.where(lane == 2, npos, 0.0)))
    prev = jnp.where(j == 0, jnp.zeros_like(vec), stats_ref[...])
    stats_ref[...] = prev + vec

    @pl.when(j == nblocks - 1)
    def _selection():
        st = stats_ref[...]
        npos_c = st[:, 2:3]
        num_neg = jnp.minimum(_NEG_POS_RATIO * npos_c, n_total - npos_c)
        has_min = jnp.sum((num_neg > 0).astype(jnp.float32))
        nnb = jnp.where(has_min > 0, jnp.sum(num_neg),
                        jnp.float32(_NEGATIVES_FOR_HARD))
        k = jnp.floor(nnb).astype(jnp.int32)

        bits = jax.lax.bitcast_convert_type(key_scr[...], jnp.int32)
        confs = conf_scr[...]

        def step(_, carry):
            lo, hi = carry
            third = jnp.maximum(jax.lax.div(hi - lo, 3), 1)
            m1 = lo + third
            m2 = jnp.maximum(hi - third, m1)
            c1 = jnp.sum((bits >= m1).astype(jnp.int32))
            c2 = jnp.sum((bits >= m2).astype(jnp.int32))
            big1 = c1 >= k
            big2 = c2 >= k
            nlo = jnp.where(big2, m2, jnp.where(big1, m1, lo))
            nhi = jnp.where(big2, hi, jnp.where(big1, m2, m1))
            return (nlo, nhi)

        lo, _ = jax.lax.fori_loop(0, 21, step,
                                  (jnp.int32(0), jnp.int32(0x7F800000)))
        gt = bits > lo
        eq = bits == lo
        sum_gt = jnp.sum(jnp.where(gt, confs, 0.0))
        cnt_gt = jnp.sum(gt.astype(jnp.float32))
        sum_eq = jnp.sum(jnp.where(eq, confs, 0.0))
        cnt_eq = jnp.sum(eq.astype(jnp.float32))
        rem = jnp.maximum(k.astype(jnp.float32) - cnt_gt, 0.0)
        rem = jnp.minimum(rem, cnt_eq)
        neg = sum_gt + rem * sum_eq / jnp.maximum(cnt_eq, 1.0)

        pos_conf = jnp.sum(st[:, 0:1])
        pos_loc = jnp.sum(st[:, 1:2])
        denom = jnp.sum(jnp.where(npos_c != 0.0, npos_c, 1.0))
        out_ref[0] = (pos_conf + neg + _ALPHA * pos_loc) / denom


def kernel(y_true, y_pred_loc, y_pred_conf):
    b, n, c = y_pred_conf.shape
    nf = y_true.shape[2]
    block_n = 512
    nblocks = (n + block_n - 1) // block_n

    yt_t = jnp.transpose(y_true, (2, 0, 1))
    xl_t = jnp.transpose(y_pred_loc, (2, 0, 1))
    xc_t = jnp.transpose(y_pred_conf, (2, 0, 1))

    out = pl.pallas_call(
        functools.partial(_body, n, block_n, nblocks),
        grid=(nblocks,),
        in_specs=[
            pl.BlockSpec((nf, b, block_n), lambda j: (0, 0, j)),
            pl.BlockSpec((4, b, block_n), lambda j: (0, 0, j)),
            pl.BlockSpec((c, b, block_n), lambda j: (0, 0, j)),
        ],
        out_specs=pl.BlockSpec(memory_space=pltpu.SMEM),
        out_shape=jax.ShapeDtypeStruct((1,), jnp.float32),
        scratch_shapes=[
            pltpu.VMEM((b, nblocks * block_n), jnp.float32),
            pltpu.VMEM((b, nblocks * block_n), jnp.float32),
            pltpu.VMEM((b, 8), jnp.float32),
        ],
    )(yt_t, xl_t, xc_t)
    return out[0]
